# Initial kernel scaffold; baseline (speedup 1.0000x reference)
import jax
import jax.numpy as jnp
from jax import lax
from jax.experimental import pallas as pl
from jax.experimental.pallas import tpu as pltpu
from jax.experimental.pallas import tpu_sc as plsc

NC, NS, L = 2, 16, 16
NW = NC * NS
CHUNK = 64
EPS = 1e-5


def _rsqrt_vec(v):
    i = lax.bitcast_convert_type(v, jnp.int32)
    y = lax.bitcast_convert_type(jnp.int32(0x5F3759DF) - lax.shift_right_arithmetic(i, 1),
                                 jnp.float32)
    half = v * 0.5
    for _ in range(4):
        y = y * (1.5 - half * y * y)
    return y


def _sc_scatter_call(me, e, p, midx, uidx, gamma, beta, B, T, K, NM, NU):
    m_per_w = (B * NM) // NW
    u_per_w = (B * NU) // NW
    n_mchunks = m_per_w // CHUNK
    n_uchunks = u_per_w // CHUNK
    inv_k = jnp.float32(1.0 / K)

    def body(me_hbm, e_hbm, p_hbm, midx_hbm, uidx_hbm, g_hbm, b_hbm, out_hbm,
             bufA, bufB, idxv, gv, bv, sem):
        wid = lax.axis_index("s") * NC + lax.axis_index("c")
        pltpu.sync_copy(g_hbm, gv)
        pltpu.sync_copy(b_hbm, bv)

        def load_idx(src_hbm, base, rows_per_batch):
            pltpu.sync_copy(src_hbm.at[pl.ds(base, CHUNK)], idxv)
            bofs = (base // rows_per_batch) * T
            for i in range(CHUNK // L):
                sl = pl.ds(i * L, L)
                idxv[sl] = idxv[sl] + bofs

        def mbody(c, carry):
            base = wid * m_per_w + c * CHUNK
            load_idx(midx_hbm, base, NM)
            pltpu.sync_copy(me_hbm.at[pl.ds(base, CHUNK)], bufA)
            pltpu.async_copy(bufA, out_hbm.at[idxv], sem).wait()
            return carry

        lax.fori_loop(0, n_mchunks, mbody, 0)

        def ubody(c, carry):
            base = wid * u_per_w + c * CHUNK
            load_idx(uidx_hbm, base, NU)
            pltpu.sync_copy(e_hbm.at[pl.ds(base, CHUNK)], bufA)
            pltpu.sync_copy(p_hbm.at[pl.ds(base, CHUNK)], bufB)

            def row(r, rcarry):
                s1 = jnp.zeros((L,), jnp.float32)
                s2 = jnp.zeros((L,), jnp.float32)
                for i in range(K // L):
                    sl = pl.ds(i * L, L)
                    xv = bufA[r, sl] + bufB[r, sl]
                    bufA[r, sl] = xv
                    s1 = s1 + xv
                    s2 = s2 + xv * xv
                mean = jnp.sum(s1) * inv_k
                var = jnp.sum(s2) * inv_k - mean * mean
                rstd = _rsqrt_vec(jnp.broadcast_to(var + EPS, (L,)))
                mvec = jnp.broadcast_to(mean, (L,))
                for i in range(K // L):
                    sl = pl.ds(i * L, L)
                    bufA[r, sl] = (bufA[r, sl] - mvec) * rstd * gv[sl] + bv[sl]
                return rcarry

            lax.fori_loop(0, CHUNK, row, 0)
            pltpu.async_copy(bufA, out_hbm.at[idxv], sem).wait()
            return carry

        lax.fori_loop(0, n_uchunks, ubody, 0)

    mesh = plsc.VectorSubcoreMesh(core_axis_name="c", subcore_axis_name="s")
    f = pl.kernel(
        body,
        out_type=jax.ShapeDtypeStruct((B * T, K), jnp.float32),
        mesh=mesh,
        scratch_types=[
            pltpu.VMEM((CHUNK, K), jnp.float32),
            pltpu.VMEM((CHUNK, K), jnp.float32),
            pltpu.VMEM((CHUNK,), jnp.int32),
            pltpu.VMEM((K,), jnp.float32),
            pltpu.VMEM((K,), jnp.float32),
            pltpu.SemaphoreType.DMA,
        ],
    )
    return f(me, e, p, midx, uidx, gamma, beta)


def kernel(encoder_output, mask_embedding, unmasked_positions, mask_id, unmask_id,
           gamma, beta):
    B, NU, K = encoder_output.shape
    NM = mask_embedding.shape[1]
    T = NM + NU
    me = mask_embedding.reshape(B * NM, K)
    e = encoder_output.reshape(B * NU, K)
    p = unmasked_positions.reshape(B * NU, K)
    midx = mask_id.reshape(B * NM)
    uidx = unmask_id.reshape(B * NU)
    out = _sc_scatter_call(me, e, p, midx, uidx, gamma, beta, B, T, K, NM, NU)
    return out.reshape(B, T, K)


# speedup vs baseline: 3.4099x; 3.4099x over previous
import jax
import jax.numpy as jnp
from jax import lax
from jax.experimental import pallas as pl
from jax.experimental.pallas import tpu as pltpu
from jax.experimental.pallas import tpu_sc as plsc

NC, NS, L = 2, 16, 16
NW = NC * NS
CHUNK = 64
EPS = 1e-5


def _lane_sum(s):
    idx = lax.iota(jnp.int32, L)
    dnums = lax.GatherDimensionNumbers(offset_dims=(), collapsed_slice_dims=(0,),
                                       start_index_map=(0,))
    for off in (8, 4, 2, 1):
        perm = lax.bitwise_xor(idx, off)
        s = s + lax.gather(s, perm[:, None], dnums, slice_sizes=(1,),
                           mode=lax.GatherScatterMode.PROMISE_IN_BOUNDS)
    return s


def _rsqrt_vec(v):
    i = lax.bitcast_convert_type(v, jnp.int32)
    y = lax.bitcast_convert_type(jnp.int32(0x5F3759DF) - lax.shift_right_arithmetic(i, 1),
                                 jnp.float32)
    half = v * 0.5
    for _ in range(4):
        y = y * (1.5 - half * y * y)
    return y


def _sc_scatter_call(me, e, p, midx, uidx, gamma, beta, B, T, K, NM, NU):
    m_per_w = (B * NM) // NW
    u_per_w = (B * NU) // NW
    n_mchunks = m_per_w // CHUNK
    n_uchunks = u_per_w // CHUNK
    inv_k = jnp.float32(1.0 / K)

    def body(me_hbm, e_hbm, p_hbm, midx_hbm, uidx_hbm, g_hbm, b_hbm, out_hbm,
             bufA, bufB, idxv, gv, bv, sem):
        wid = lax.axis_index("s") * NC + lax.axis_index("c")
        pltpu.sync_copy(g_hbm, gv)
        pltpu.sync_copy(b_hbm, bv)

        def load_idx(src_hbm, base, rows_per_batch):
            pltpu.sync_copy(src_hbm.at[pl.ds(base, CHUNK)], idxv)
            bofs = (base // rows_per_batch) * T
            for i in range(CHUNK // L):
                sl = pl.ds(i * L, L)
                idxv[sl] = idxv[sl] + bofs

        def mbody(c, carry):
            base = wid * m_per_w + c * CHUNK
            load_idx(midx_hbm, base, NM)
            pltpu.sync_copy(me_hbm.at[pl.ds(base, CHUNK)], bufA)
            pltpu.async_copy(bufA, out_hbm.at[idxv], sem).wait()
            return carry

        lax.fori_loop(0, n_mchunks, mbody, 0)

        def ubody(c, carry):
            base = wid * u_per_w + c * CHUNK
            load_idx(uidx_hbm, base, NU)
            pltpu.sync_copy(e_hbm.at[pl.ds(base, CHUNK)], bufA)
            pltpu.sync_copy(p_hbm.at[pl.ds(base, CHUNK)], bufB)

            def row(r, rcarry):
                s1 = jnp.zeros((L,), jnp.float32)
                s2 = jnp.zeros((L,), jnp.float32)
                for i in range(K // L):
                    sl = pl.ds(i * L, L)
                    xv = bufA[r, sl] + bufB[r, sl]
                    bufA[r, sl] = xv
                    s1 = s1 + xv
                    s2 = s2 + xv * xv
                mvec = _lane_sum(s1) * inv_k
                var = _lane_sum(s2) * inv_k - mvec * mvec
                rstd = _rsqrt_vec(var + EPS)
                for i in range(K // L):
                    sl = pl.ds(i * L, L)
                    bufA[r, sl] = (bufA[r, sl] - mvec) * rstd * gv[sl] + bv[sl]
                return rcarry

            lax.fori_loop(0, CHUNK, row, 0)
            pltpu.async_copy(bufA, out_hbm.at[idxv], sem).wait()
            return carry

        lax.fori_loop(0, n_uchunks, ubody, 0)

    mesh = plsc.VectorSubcoreMesh(core_axis_name="c", subcore_axis_name="s")
    f = pl.kernel(
        body,
        out_type=jax.ShapeDtypeStruct((B * T, K), jnp.float32),
        mesh=mesh,
        scratch_types=[
            pltpu.VMEM((CHUNK, K), jnp.float32),
            pltpu.VMEM((CHUNK, K), jnp.float32),
            pltpu.VMEM((CHUNK,), jnp.int32),
            pltpu.VMEM((K,), jnp.float32),
            pltpu.VMEM((K,), jnp.float32),
            pltpu.SemaphoreType.DMA,
        ],
    )
    return f(me, e, p, midx, uidx, gamma, beta)


def kernel(encoder_output, mask_embedding, unmasked_positions, mask_id, unmask_id,
           gamma, beta):
    B, NU, K = encoder_output.shape
    NM = mask_embedding.shape[1]
    T = NM + NU
    me = mask_embedding.reshape(B * NM, K)
    e = encoder_output.reshape(B * NU, K)
    p = unmasked_positions.reshape(B * NU, K)
    midx = mask_id.reshape(B * NM)
    uidx = unmask_id.reshape(B * NU)
    out = _sc_scatter_call(me, e, p, midx, uidx, gamma, beta, B, T, K, NM, NU)
    return out.reshape(B, T, K)


# speedup vs baseline: 3.8011x; 1.1147x over previous
import jax
import jax.numpy as jnp
from jax import lax
from jax.experimental import pallas as pl
from jax.experimental.pallas import tpu as pltpu
from jax.experimental.pallas import tpu_sc as plsc

NC, NS, L = 2, 16, 16
NW = NC * NS
CHUNK = 32
NSLOT = 6
EPS = 1e-5


def _lane_sum(s):
    idx = lax.iota(jnp.int32, L)
    dnums = lax.GatherDimensionNumbers(offset_dims=(), collapsed_slice_dims=(0,),
                                       start_index_map=(0,))
    for off in (8, 4, 2, 1):
        perm = lax.bitwise_xor(idx, off)
        s = s + lax.gather(s, perm[:, None], dnums, slice_sizes=(1,),
                           mode=lax.GatherScatterMode.PROMISE_IN_BOUNDS)
    return s


def _rsqrt_vec(v):
    i = lax.bitcast_convert_type(v, jnp.int32)
    y = lax.bitcast_convert_type(jnp.int32(0x5F3759DF) - lax.shift_right_arithmetic(i, 1),
                                 jnp.float32)
    half = v * 0.5
    for _ in range(4):
        y = y * (1.5 - half * y * y)
    return y


def _sc_scatter_call(me, e, p, midx, uidx, gamma, beta, B, T, K, NM, NU):
    m_per_w = (B * NM) // NW
    u_per_w = (B * NU) // NW
    n_mchunks = m_per_w // CHUNK
    n_uchunks = u_per_w // CHUNK
    n_mgroups = n_mchunks // NSLOT
    n_ugroups = n_uchunks // 2
    inv_k = jnp.float32(1.0 / K)

    def body(me_hbm, e_hbm, p_hbm, midx_hbm, uidx_hbm, g_hbm, b_hbm, out_hbm,
             D, idxb, gv, bv, *sems):
        sem_g = sems[:NSLOT]
        sem_s = sems[NSLOT:]
        wid = lax.axis_index("s") * NC + lax.axis_index("c")
        pltpu.sync_copy(g_hbm, gv)
        pltpu.sync_copy(b_hbm, bv)

        def adjust_idx(slot, base, rows_per_batch):
            bofs = (base // rows_per_batch) * T
            for i in range(CHUNK // L):
                sl = pl.ds(i * L, L)
                idxb[slot, sl] = idxb[slot, sl] + bofs

        def m_base(c):
            return wid * m_per_w + c * CHUNK

        def m_gather(c, slot):
            base = m_base(c)
            return (pltpu.make_async_copy(midx_hbm.at[pl.ds(base, CHUNK)],
                                          idxb.at[slot], sem_g[slot]),
                    pltpu.make_async_copy(me_hbm.at[pl.ds(base, CHUNK)],
                                          D.at[slot], sem_g[slot]))

        def m_scatter(slot):
            return pltpu.make_async_copy(D.at[slot], out_hbm.at[idxb.at[slot]],
                                         sem_s[slot])

        for b in range(NSLOT):
            for d in m_gather(b, b):
                d.start()

        def mgroup(g, carry):
            for b in range(NSLOT):
                c = g * NSLOT + b
                for d in m_gather(c, b):
                    d.wait()
                adjust_idx(b, m_base(c), NM)
                m_scatter(b).start()
            for b in range(NSLOT):
                c = g * NSLOT + b

                @pl.when(c + NSLOT < n_mchunks)
                def _():
                    m_scatter(b).wait()
                    for d in m_gather(c + NSLOT, b):
                        d.start()

            return carry

        lax.fori_loop(0, n_mgroups, mgroup, 0)
        for b in range(NSLOT):
            m_scatter(b).wait()

        def u_base(c):
            return wid * u_per_w + c * CHUNK

        def u_gather(c, b):
            base = u_base(c)
            return (pltpu.make_async_copy(uidx_hbm.at[pl.ds(base, CHUNK)],
                                          idxb.at[2 * b], sem_g[2 * b]),
                    pltpu.make_async_copy(e_hbm.at[pl.ds(base, CHUNK)],
                                          D.at[2 * b], sem_g[2 * b]),
                    pltpu.make_async_copy(p_hbm.at[pl.ds(base, CHUNK)],
                                          D.at[2 * b + 1], sem_g[2 * b]))

        def u_scatter(b):
            return pltpu.make_async_copy(D.at[2 * b], out_hbm.at[idxb.at[2 * b]],
                                         sem_s[2 * b])

        for b in range(2):
            for d in u_gather(b, b):
                d.start()

        def ugroup(g, carry):
            for b in range(2):
                c = 2 * g + b
                for d in u_gather(c, b):
                    d.wait()
                adjust_idx(2 * b, u_base(c), NU)

                def row(r, rcarry):
                    s1 = jnp.zeros((L,), jnp.float32)
                    s2 = jnp.zeros((L,), jnp.float32)
                    for i in range(K // L):
                        sl = pl.ds(i * L, L)
                        xv = D[2 * b, r, sl] + D[2 * b + 1, r, sl]
                        D[2 * b, r, sl] = xv
                        s1 = s1 + xv
                        s2 = s2 + xv * xv
                    mvec = _lane_sum(s1) * inv_k
                    var = _lane_sum(s2) * inv_k - mvec * mvec
                    rstd = _rsqrt_vec(var + EPS)
                    for i in range(K // L):
                        sl = pl.ds(i * L, L)
                        D[2 * b, r, sl] = ((D[2 * b, r, sl] - mvec) * rstd
                                           * gv[sl] + bv[sl])
                    return rcarry

                lax.fori_loop(0, CHUNK, row, 0)
                u_scatter(b).start()
            for b in range(2):
                c = 2 * g + b

                @pl.when(c + 2 < n_uchunks)
                def _():
                    u_scatter(b).wait()
                    for d in u_gather(c + 2, b):
                        d.start()

            return carry

        lax.fori_loop(0, n_ugroups, ugroup, 0)
        for b in range(2):
            u_scatter(b).wait()

    mesh = plsc.VectorSubcoreMesh(core_axis_name="c", subcore_axis_name="s")
    f = pl.kernel(
        body,
        out_type=jax.ShapeDtypeStruct((B * T, K), jnp.float32),
        mesh=mesh,
        scratch_types=[
            pltpu.VMEM((NSLOT, CHUNK, K), jnp.float32),
            pltpu.VMEM((NSLOT, CHUNK), jnp.int32),
            pltpu.VMEM((K,), jnp.float32),
            pltpu.VMEM((K,), jnp.float32),
        ] + [pltpu.SemaphoreType.DMA] * (2 * NSLOT),
    )
    return f(me, e, p, midx, uidx, gamma, beta)


def kernel(encoder_output, mask_embedding, unmasked_positions, mask_id, unmask_id,
           gamma, beta):
    B, NU, K = encoder_output.shape
    NM = mask_embedding.shape[1]
    T = NM + NU
    me = mask_embedding.reshape(B * NM, K)
    e = encoder_output.reshape(B * NU, K)
    p = unmasked_positions.reshape(B * NU, K)
    midx = mask_id.reshape(B * NM)
    uidx = unmask_id.reshape(B * NU)
    out = _sc_scatter_call(me, e, p, midx, uidx, gamma, beta, B, T, K, NM, NU)
    return out.reshape(B, T, K)


# speedup vs baseline: 4.0929x; 1.0768x over previous
import jax
import jax.numpy as jnp
from jax import lax
from jax.experimental import pallas as pl
from jax.experimental.pallas import tpu as pltpu
from jax.experimental.pallas import tpu_sc as plsc

NC, NS, L = 2, 16, 16
NW = NC * NS
CHUNK = 32
NSLOT = 6
EPS = 1e-5


def _lane_sum(s):
    idx = lax.iota(jnp.int32, L)
    dnums = lax.GatherDimensionNumbers(offset_dims=(), collapsed_slice_dims=(0,),
                                       start_index_map=(0,))
    for off in (8, 4, 2, 1):
        perm = lax.bitwise_xor(idx, off)
        s = s + lax.gather(s, perm[:, None], dnums, slice_sizes=(1,),
                           mode=lax.GatherScatterMode.PROMISE_IN_BOUNDS)
    return s


def _rsqrt_vec(v):
    i = lax.bitcast_convert_type(v, jnp.int32)
    y = lax.bitcast_convert_type(jnp.int32(0x5F3759DF) - lax.shift_right_arithmetic(i, 1),
                                 jnp.float32)
    half = v * 0.5
    for _ in range(4):
        y = y * (1.5 - half * y * y)
    return y


def _sc_scatter_call(me, e, p, midx, uidx, gamma, beta, B, T, K, NM, NU):
    m_per_w = (B * NM) // NW
    u_per_w = (B * NU) // NW
    n_mchunks = m_per_w // CHUNK
    n_uchunks = u_per_w // CHUNK
    n_mgroups = n_mchunks // NSLOT
    n_ugroups = (n_uchunks - 1) // 3
    inv_k = jnp.float32(1.0 / K)

    def body(me_hbm, e_hbm, p_hbm, midx_hbm, uidx_hbm, g_hbm, b_hbm, out_hbm,
             D, idxm, idxu, gv, bv, *sems):
        sem_g = sems[:NSLOT]
        sem_s = sems[NSLOT:2 * NSLOT]
        sem_ug = sems[2 * NSLOT:2 * NSLOT + 3]
        sem_us = sems[2 * NSLOT + 3:]
        wid = lax.axis_index("s") * NC + lax.axis_index("c")
        pltpu.sync_copy(g_hbm, gv)
        pltpu.sync_copy(b_hbm, bv)

        def adjust_idx(ref, slot, base, rows_per_batch):
            bofs = (base // rows_per_batch) * T
            for i in range(CHUNK // L):
                sl = pl.ds(i * L, L)
                ref[slot, sl] = ref[slot, sl] + bofs

        def m_base(c):
            return wid * m_per_w + c * CHUNK

        def m_gather(c, slot):
            base = m_base(c)
            return (pltpu.make_async_copy(midx_hbm.at[pl.ds(base, CHUNK)],
                                          idxm.at[slot], sem_g[slot]),
                    pltpu.make_async_copy(me_hbm.at[pl.ds(base, CHUNK)],
                                          D.at[slot], sem_g[slot]))

        def m_scatter(slot):
            return pltpu.make_async_copy(D.at[slot], out_hbm.at[idxm.at[slot]],
                                         sem_s[slot])

        for b in range(NSLOT):
            for d in m_gather(b, b):
                d.start()

        def mgroup(g, carry):
            for b in range(NSLOT):
                c = g * NSLOT + b
                for d in m_gather(c, b):
                    d.wait()
                adjust_idx(idxm, b, m_base(c), NM)
                m_scatter(b).start()
                sl = (b + 3) % NSLOT
                cprev = c - 3

                @pl.when(cprev >= 0)
                def _():
                    m_scatter(sl).wait()

                @pl.when(jnp.logical_and(cprev >= 0, c + 3 < n_mchunks))
                def _():
                    for d in m_gather(c + 3, sl):
                        d.start()

            return carry

        lax.fori_loop(0, n_mgroups, mgroup, 0)
        for b in range(3):
            m_scatter((n_mchunks - 3 + b) % NSLOT).wait()

        def u_base(c):
            return wid * u_per_w + c * CHUNK

        def u_gather(c, pr):
            base = u_base(c)
            return (pltpu.make_async_copy(uidx_hbm.at[pl.ds(base, CHUNK)],
                                          idxu.at[pr], sem_ug[pr]),
                    pltpu.make_async_copy(e_hbm.at[pl.ds(base, CHUNK)],
                                          D.at[2 * pr], sem_ug[pr]),
                    pltpu.make_async_copy(p_hbm.at[pl.ds(base, CHUNK)],
                                          D.at[2 * pr + 1], sem_ug[pr]))

        def u_scatter(pr):
            return pltpu.make_async_copy(D.at[2 * pr], out_hbm.at[idxu.at[pr]],
                                         sem_us[pr])

        def u_compute(c, pr):
            adjust_idx(idxu, pr, u_base(c), NU)

            def row(r, rcarry):
                s1 = jnp.zeros((L,), jnp.float32)
                s2 = jnp.zeros((L,), jnp.float32)
                for i in range(K // L):
                    sl = pl.ds(i * L, L)
                    xv = D[2 * pr, r, sl] + D[2 * pr + 1, r, sl]
                    D[2 * pr, r, sl] = xv
                    s1 = s1 + xv
                    s2 = s2 + xv * xv
                mvec = _lane_sum(s1) * inv_k
                var = _lane_sum(s2) * inv_k - mvec * mvec
                rstd = _rsqrt_vec(var + EPS)
                for i in range(K // L):
                    sl = pl.ds(i * L, L)
                    D[2 * pr, r, sl] = ((D[2 * pr, r, sl] - mvec) * rstd
                                        * gv[sl] + bv[sl])
                return rcarry

            lax.fori_loop(0, CHUNK, row, 0)

        def u_step(c, pr):
            for d in u_gather(c, pr):
                d.wait()
            npr = (pr + 1) % 3
            cnext = c + 1

            @pl.when(c - 2 >= 0)
            def _():
                u_scatter(npr).wait()

            @pl.when(jnp.logical_and(cnext >= 2, cnext < n_uchunks))
            def _():
                for d in u_gather(cnext, npr):
                    d.start()

            u_compute(c, pr)
            u_scatter(pr).start()

        for c in range(2):
            for d in u_gather(c, c):
                d.start()

        def ugroup(g, carry):
            for b in range(3):
                u_step(g * 3 + b, b)
            return carry

        lax.fori_loop(0, n_ugroups, ugroup, 0)
        u_step(n_uchunks - 1, (n_uchunks - 1) % 3)
        for j in range(2):
            u_scatter((n_uchunks - 2 + j) % 3).wait()

    mesh = plsc.VectorSubcoreMesh(core_axis_name="c", subcore_axis_name="s")
    f = pl.kernel(
        body,
        out_type=jax.ShapeDtypeStruct((B * T, K), jnp.float32),
        mesh=mesh,
        scratch_types=[
            pltpu.VMEM((NSLOT, CHUNK, K), jnp.float32),
            pltpu.VMEM((NSLOT, CHUNK), jnp.int32),
            pltpu.VMEM((3, CHUNK), jnp.int32),
            pltpu.VMEM((K,), jnp.float32),
            pltpu.VMEM((K,), jnp.float32),
        ] + [pltpu.SemaphoreType.DMA] * (2 * NSLOT + 6),
    )
    return f(me, e, p, midx, uidx, gamma, beta)


def kernel(encoder_output, mask_embedding, unmasked_positions, mask_id, unmask_id,
           gamma, beta):
    B, NU, K = encoder_output.shape
    NM = mask_embedding.shape[1]
    T = NM + NU
    me = mask_embedding.reshape(B * NM, K)
    e = encoder_output.reshape(B * NU, K)
    p = unmasked_positions.reshape(B * NU, K)
    midx = mask_id.reshape(B * NM)
    uidx = unmask_id.reshape(B * NU)
    out = _sc_scatter_call(me, e, p, midx, uidx, gamma, beta, B, T, K, NM, NU)
    return out.reshape(B, T, K)


# speedup vs baseline: 4.6232x; 1.1296x over previous
import jax
import jax.numpy as jnp
from jax import lax
from jax.experimental import pallas as pl
from jax.experimental.pallas import tpu as pltpu
from jax.experimental.pallas import tpu_sc as plsc

NC, NS, L = 2, 16, 16
NW = NC * NS
CHUNK = 32
NSLOT = 6
EPS = 1e-5


def _lane_sum(s):
    idx = lax.iota(jnp.int32, L)
    dnums = lax.GatherDimensionNumbers(offset_dims=(), collapsed_slice_dims=(0,),
                                       start_index_map=(0,))
    for off in (8, 4, 2, 1):
        perm = lax.bitwise_xor(idx, off)
        s = s + lax.gather(s, perm[:, None], dnums, slice_sizes=(1,),
                           mode=lax.GatherScatterMode.PROMISE_IN_BOUNDS)
    return s


def _rsqrt_vec(v):
    i = lax.bitcast_convert_type(v, jnp.int32)
    y = lax.bitcast_convert_type(jnp.int32(0x5F3759DF) - lax.shift_right_arithmetic(i, 1),
                                 jnp.float32)
    half = v * 0.5
    for _ in range(3):
        y = y * (1.5 - half * y * y)
    return y


def _sc_scatter_call(me, e, p, midx, uidx, gamma, beta, B, T, K, NM, NU):
    m_per_w = (B * NM) // NW
    u_per_w = (B * NU) // NW
    n_mchunks = m_per_w // CHUNK
    n_uchunks = u_per_w // CHUNK
    n_mgroups = n_mchunks // NSLOT
    n_ugroups = (n_uchunks - 1) // 3
    inv_k = jnp.float32(1.0 / K)

    def body(me_hbm, e_hbm, p_hbm, midx_hbm, uidx_hbm, g_hbm, b_hbm, out_hbm,
             D, idxm, idxu, gv, bv, *sems):
        sem_g = sems[:NSLOT]
        sem_s = sems[NSLOT:2 * NSLOT]
        sem_ug = sems[2 * NSLOT:2 * NSLOT + 3]
        sem_us = sems[2 * NSLOT + 3:]
        wid = lax.axis_index("s") * NC + lax.axis_index("c")
        pltpu.sync_copy(g_hbm, gv)
        pltpu.sync_copy(b_hbm, bv)

        def adjust_idx(ref, slot, base, rows_per_batch):
            bofs = (base // rows_per_batch) * T
            for i in range(CHUNK // L):
                sl = pl.ds(i * L, L)
                ref[slot, sl] = ref[slot, sl] + bofs

        def m_base(c):
            return wid * m_per_w + c * CHUNK

        def m_gather(c, slot):
            base = m_base(c)
            return (pltpu.make_async_copy(midx_hbm.at[pl.ds(base, CHUNK)],
                                          idxm.at[slot], sem_g[slot]),
                    pltpu.make_async_copy(me_hbm.at[pl.ds(base, CHUNK)],
                                          D.at[slot], sem_g[slot]))

        def m_scatter(slot):
            return pltpu.make_async_copy(D.at[slot], out_hbm.at[idxm.at[slot]],
                                         sem_s[slot])

        for b in range(NSLOT):
            for d in m_gather(b, b):
                d.start()

        def mgroup(g, carry):
            for b in range(NSLOT):
                c = g * NSLOT + b
                for d in m_gather(c, b):
                    d.wait()
                adjust_idx(idxm, b, m_base(c), NM)
                m_scatter(b).start()
                sl = (b + 3) % NSLOT
                cprev = c - 3

                @pl.when(cprev >= 0)
                def _():
                    m_scatter(sl).wait()

                @pl.when(jnp.logical_and(cprev >= 0, c + 3 < n_mchunks))
                def _():
                    for d in m_gather(c + 3, sl):
                        d.start()

            return carry

        lax.fori_loop(0, n_mgroups, mgroup, 0)
        for b in range(3):
            m_scatter((n_mchunks - 3 + b) % NSLOT).wait()

        def u_base(c):
            return wid * u_per_w + c * CHUNK

        def u_gather(c, pr):
            base = u_base(c)
            return (pltpu.make_async_copy(uidx_hbm.at[pl.ds(base, CHUNK)],
                                          idxu.at[pr], sem_ug[pr]),
                    pltpu.make_async_copy(e_hbm.at[pl.ds(base, CHUNK)],
                                          D.at[2 * pr], sem_ug[pr]),
                    pltpu.make_async_copy(p_hbm.at[pl.ds(base, CHUNK)],
                                          D.at[2 * pr + 1], sem_ug[pr]))

        def u_scatter(pr):
            return pltpu.make_async_copy(D.at[2 * pr], out_hbm.at[idxu.at[pr]],
                                         sem_us[pr])

        def u_compute(c, pr):
            adjust_idx(idxu, pr, u_base(c), NU)

            def rowpair(rr, rcarry):
                rows = (2 * rr, 2 * rr + 1)
                s1 = [jnp.zeros((L,), jnp.float32) for _ in rows]
                s2 = [jnp.zeros((L,), jnp.float32) for _ in rows]
                for i in range(K // L):
                    sl = pl.ds(i * L, L)
                    for j, r in enumerate(rows):
                        xv = D[2 * pr, r, sl] + D[2 * pr + 1, r, sl]
                        D[2 * pr, r, sl] = xv
                        s1[j] = s1[j] + xv
                        s2[j] = s2[j] + xv * xv
                mvec = [None, None]
                rstd = [None, None]
                for j in range(2):
                    mvec[j] = _lane_sum(s1[j]) * inv_k
                    var = _lane_sum(s2[j]) * inv_k - mvec[j] * mvec[j]
                    rstd[j] = _rsqrt_vec(var + EPS)
                for i in range(K // L):
                    sl = pl.ds(i * L, L)
                    gsl = gv[sl]
                    bsl = bv[sl]
                    for j, r in enumerate(rows):
                        D[2 * pr, r, sl] = ((D[2 * pr, r, sl] - mvec[j])
                                            * rstd[j] * gsl + bsl)
                return rcarry

            lax.fori_loop(0, CHUNK // 2, rowpair, 0)

        def u_step(c, pr):
            for d in u_gather(c, pr):
                d.wait()
            npr = (pr + 1) % 3
            cnext = c + 1

            @pl.when(c - 2 >= 0)
            def _():
                u_scatter(npr).wait()

            @pl.when(jnp.logical_and(cnext >= 2, cnext < n_uchunks))
            def _():
                for d in u_gather(cnext, npr):
                    d.start()

            u_compute(c, pr)
            u_scatter(pr).start()

        for c in range(2):
            for d in u_gather(c, c):
                d.start()

        def ugroup(g, carry):
            for b in range(3):
                u_step(g * 3 + b, b)
            return carry

        lax.fori_loop(0, n_ugroups, ugroup, 0)
        u_step(n_uchunks - 1, (n_uchunks - 1) % 3)
        for j in range(2):
            u_scatter((n_uchunks - 2 + j) % 3).wait()

    mesh = plsc.VectorSubcoreMesh(core_axis_name="c", subcore_axis_name="s")
    f = pl.kernel(
        body,
        out_type=jax.ShapeDtypeStruct((B * T, K), jnp.float32),
        mesh=mesh,
        scratch_types=[
            pltpu.VMEM((NSLOT, CHUNK, K), jnp.float32),
            pltpu.VMEM((NSLOT, CHUNK), jnp.int32),
            pltpu.VMEM((3, CHUNK), jnp.int32),
            pltpu.VMEM((K,), jnp.float32),
            pltpu.VMEM((K,), jnp.float32),
        ] + [pltpu.SemaphoreType.DMA] * (2 * NSLOT + 6),
    )
    return f(me, e, p, midx, uidx, gamma, beta)


def kernel(encoder_output, mask_embedding, unmasked_positions, mask_id, unmask_id,
           gamma, beta):
    B, NU, K = encoder_output.shape
    NM = mask_embedding.shape[1]
    T = NM + NU
    me = mask_embedding.reshape(B * NM, K)
    e = encoder_output.reshape(B * NU, K)
    p = unmasked_positions.reshape(B * NU, K)
    midx = mask_id.reshape(B * NM)
    uidx = unmask_id.reshape(B * NU)
    out = _sc_scatter_call(me, e, p, midx, uidx, gamma, beta, B, T, K, NM, NU)
    return out.reshape(B, T, K)
